# baseline (speedup 1.0000x reference)
import functools

import jax
import jax.numpy as jnp
from jax import lax
from jax.experimental import pallas as pl
from jax.experimental.pallas import tpu as pltpu
from jax.experimental.pallas import tpu_sc as plsc

_NC = 2
_NS = 16
_LANES = 16


@functools.lru_cache(maxsize=None)
def _make_scatter(n, d, e_pad):
    nw = _NC * _NS
    chunk = 80
    nbuf = 4
    assert e_pad % (nw * chunk) == 0
    cpw = e_pad // (nw * chunk)
    epw = cpw * chunk
    nsup = cpw // nbuf
    assert cpw % (2 * nbuf) == 0 and epw % 8 == 0
    rpt = (n // _NS) // 8 * 8
    rextra = n - _NS * rpt
    assert rextra % 8 == 0
    zrows = 16
    assert rpt % zrows == 0 and rextra <= zrows
    nacc = n + 8
    mesh = plsc.VectorSubcoreMesh(core_axis_name="c", subcore_axis_name="s")

    @functools.partial(
        pl.kernel,
        mesh=mesh,
        out_type=jax.ShapeDtypeStruct((_NC, n, d), jnp.float32),
        scratch_types=(
            [pltpu.VMEM((zrows, d), jnp.float32)]
            + [pltpu.VMEM((chunk,), jnp.int32) for _ in range(2 * nbuf)]
            + [pltpu.VMEM((chunk,), jnp.int32) for _ in range(2 * nbuf)]
            + [pltpu.VMEM((chunk, d), jnp.float32) for _ in range(nbuf)]
            + [pltpu.SemaphoreType.DMA for _ in range(4)]
            + [pltpu.VMEM_SHARED((nacc, d), jnp.float32)]
        ),
    )
    def scatter_kernel(h_hbm, src_hbm, dst_hbm, out_hbm, zbuf, *rest):
        srcb = (rest[0:nbuf], rest[nbuf:2 * nbuf])
        dstb = (rest[2 * nbuf:3 * nbuf], rest[3 * nbuf:4 * nbuf])
        rows = rest[4 * nbuf:5 * nbuf]
        isem = rest[5 * nbuf:5 * nbuf + 2]
        gsem = rest[5 * nbuf + 2]
        ssem = rest[5 * nbuf + 3]
        acc_sh = rest[5 * nbuf + 4]
        cid = lax.axis_index("c")
        sid = lax.axis_index("s")
        wid = sid * _NC + cid
        e0 = wid * epw

        def idx_issue(st, s):
            for b in range(nbuf):
                off = e0 + (s * nbuf + b) * chunk
                pltpu.async_copy(src_hbm.at[pl.ds(off, chunk)], srcb[st][b], isem[st])
                pltpu.async_copy(dst_hbm.at[pl.ds(off, chunk)], dstb[st][b], isem[st])

        def idx_wait(st, s):
            for b in range(nbuf):
                off = e0 + (s * nbuf + b) * chunk
                pltpu.make_async_copy(src_hbm.at[pl.ds(off, chunk)],
                                      srcb[st][b], isem[st]).wait()
                pltpu.make_async_copy(dst_hbm.at[pl.ds(off, chunk)],
                                      dstb[st][b], isem[st]).wait()

        def gather_issue(st, b):
            pltpu.async_copy(h_hbm.at[srcb[st][b]], rows[b], gsem)

        def gather_wait(st, b):
            pltpu.make_async_copy(h_hbm.at[srcb[st][b]], rows[b], gsem).wait()

        def scatter_issue(st, b):
            pltpu.async_copy(rows[b], acc_sh.at[dstb[st][b]], ssem, add=True)

        def scatter_wait(st, b):
            pltpu.make_async_copy(rows[b], acc_sh.at[dstb[st][b]], ssem).wait()

        idx_issue(0, 0)

        def zstore(t, carry):
            r = t // (d // _LANES)
            c16 = (t % (d // _LANES)) * _LANES
            zbuf[r, pl.ds(c16, _LANES)] = jnp.zeros((_LANES,), jnp.float32)
            return carry
        lax.fori_loop(0, zrows * (d // _LANES), zstore, 0)
        row0 = sid * rpt
        def zcopy(j, carry):
            pltpu.sync_copy(zbuf, acc_sh.at[pl.ds(row0 + j * zrows, zrows)])
            return carry
        lax.fori_loop(0, rpt // zrows, zcopy, 0)
        @pl.when(sid == _NS - 1)
        def _ztail():
            pltpu.sync_copy(zbuf.at[pl.ds(0, rextra)],
                            acc_sh.at[pl.ds(_NS * rpt, rextra)])
        plsc.subcore_barrier()

        def half(st, s):
            idx_wait(st, s)
            idx_issue(1 - st, s + 1)
            for b in range(nbuf):
                gather_issue(st, b)
            for b in range(nbuf):
                gather_wait(st, b)
            for b in range(nbuf):
                scatter_issue(st, b)
            for b in range(nbuf):
                scatter_wait(st, b)

        def step(i, carry):
            half(0, 2 * i)
            half(1, 2 * i + 1)
            return carry
        lax.fori_loop(0, nsup // 2, step, 0)
        idx_wait(0, nsup)
        plsc.subcore_barrier()

        pltpu.sync_copy(acc_sh.at[pl.ds(row0, rpt)], out_hbm.at[cid].at[pl.ds(row0, rpt)])
        @pl.when(sid == _NS - 1)
        def _():
            pltpu.sync_copy(acc_sh.at[pl.ds(_NS * rpt, rextra)],
                            out_hbm.at[cid].at[pl.ds(_NS * rpt, rextra)])

    return scatter_kernel


@functools.lru_cache(maxsize=None)
def _make_dense(n, d_in, d, relu_out):
    def body(h_ref, a0_ref, a1_ref, w1_ref, b1_ref, w2_ref, b2_ref,
             g_ref, bt_ref, o_ref):
        z = h_ref[...] + a0_ref[...] + a1_ref[...]
        z = jnp.dot(z, w1_ref[...], preferred_element_type=jnp.float32) + b1_ref[...]
        z = jnp.maximum(z, 0.0)
        z = jnp.dot(z, w2_ref[...], preferred_element_type=jnp.float32) + b2_ref[...]
        mu = jnp.mean(z, axis=0, keepdims=True)
        var = jnp.mean((z - mu) * (z - mu), axis=0, keepdims=True)
        z = g_ref[...] * (z - mu) * lax.rsqrt(var + 1e-5) + bt_ref[...]
        if relu_out:
            z = jnp.maximum(z, 0.0)
        o_ref[...] = z

    return pl.pallas_call(
        body,
        out_shape=jax.ShapeDtypeStruct((n, d), jnp.float32),
    )


def kernel(nodes, edge_indexs, graph_indicators,
           W1_0, b1_0, W2_0, b2_0, gamma_0, beta_0,
           W1_1, b1_1, W2_1, b2_1, gamma_1, beta_1,
           W1_2, b1_2, W2_2, b2_2, gamma_2, beta_2):
    del graph_indicators
    n, d = nodes.shape
    e = edge_indexs.shape[1]
    chunk, nw, nbuf = 80, _NC * _NS, 4
    gran = chunk * nw * nbuf
    e_pad = -(-e // gran) * gran
    npad = e_pad - e + nbuf * chunk
    src = jnp.concatenate([edge_indexs[0], jnp.zeros((npad,), jnp.int32)])
    dst = jnp.concatenate([edge_indexs[1], jnp.full((npad,), n, jnp.int32)])
    params = [
        (W1_0, b1_0, W2_0, b2_0, gamma_0, beta_0),
        (W1_1, b1_1, W2_1, b2_1, gamma_1, beta_1),
        (W1_2, b1_2, W2_2, b2_2, gamma_2, beta_2),
    ]
    scatter = _make_scatter(n, d, e_pad)
    h = nodes
    for layer, (w1, b1, w2, b2, g, bt) in enumerate(params):
        agg = scatter(h, src, dst)
        dense = _make_dense(n, w1.shape[0], d, layer < len(params) - 1)
        h = dense(h, agg[0], agg[1], w1, b1.reshape(1, d), w2, b2.reshape(1, d),
                  g.reshape(1, d), bt.reshape(1, d))
    return h


# speedup vs baseline: 1.0666x; 1.0666x over previous
import functools

import jax
import jax.numpy as jnp
from jax import lax
from jax.experimental import pallas as pl
from jax.experimental.pallas import tpu as pltpu
from jax.experimental.pallas import tpu_sc as plsc

_NC = 2
_NS = 16
_LANES = 16


@functools.lru_cache(maxsize=None)
def _make_scatter(n, d, e_pad):
    nw = _NC * _NS
    chunk = 128
    assert e_pad % (nw * chunk) == 0
    cpw = e_pad // (nw * chunk)
    epw = cpw * chunk
    assert cpw % 16 == 0 and epw % 8 == 0
    hcp = cpw // 2
    rpt = (n // _NS) // 8 * 8
    rextra = n - _NS * rpt
    assert rextra % 8 == 0
    zsrc = min(chunk, rpt)
    nacc = n + 8
    mesh = plsc.VectorSubcoreMesh(core_axis_name="c", subcore_axis_name="s")

    @functools.partial(
        pl.kernel,
        mesh=mesh,
        out_type=jax.ShapeDtypeStruct((_NC, n, d), jnp.float32),
        scratch_types=[
            pltpu.VMEM((hcp * chunk,), jnp.int32),
            pltpu.VMEM((hcp, chunk), jnp.int32),
            pltpu.VMEM((chunk, d), jnp.float32),
            pltpu.VMEM((chunk, d), jnp.float32),
            pltpu.SemaphoreType.DMA,
            pltpu.SemaphoreType.DMA,
            pltpu.VMEM_SHARED((nacc, d), jnp.float32),
        ],
    )
    def scatter_kernel(h_hbm, src_hbm, dst_hbm, out_hbm,
                       src_blk, dst_blk, r0, r1, gsem, ssem, acc_sh):
        rows = (r0, r1)
        cid = lax.axis_index("c")
        sid = lax.axis_index("s")
        wid = sid * _NC + cid

        def gather_issue(b, t):
            pltpu.async_copy(h_hbm.at[src_blk.at[pl.ds(t * chunk, chunk)]],
                             rows[b], gsem)

        def gather_wait(b, t):
            pltpu.make_async_copy(h_hbm.at[src_blk.at[pl.ds(t * chunk, chunk)]],
                                  rows[b], gsem).wait()

        def scatter_issue(b, t):
            pltpu.async_copy(rows[b], acc_sh.at[dst_blk.at[t]], ssem, add=True)

        def scatter_wait(b, t):
            pltpu.make_async_copy(rows[b], acc_sh.at[dst_blk.at[t]], ssem).wait()

        def idx_load(ph):
            pltpu.sync_copy(
                src_hbm.at[pl.ds((wid * cpw + ph * hcp) * chunk, hcp * chunk)],
                src_blk)
            pltpu.sync_copy(dst_hbm.at[pl.ds(wid * cpw + ph * hcp, hcp)], dst_blk)

        def zstore(t, carry):
            r = t // (d // _LANES)
            c16 = (t % (d // _LANES)) * _LANES
            r0[r, pl.ds(c16, _LANES)] = jnp.zeros((_LANES,), jnp.float32)
            return carry
        lax.fori_loop(0, zsrc * (d // _LANES), zstore, 0)
        row0 = sid * rpt
        nz_full = rpt // zsrc
        def zcopy(j, carry):
            pltpu.sync_copy(r0.at[pl.ds(0, zsrc)],
                            acc_sh.at[pl.ds(row0 + j * zsrc, zsrc)])
            return carry
        lax.fori_loop(0, nz_full, zcopy, 0)
        zrem = rpt - nz_full * zsrc
        if zrem:
            pltpu.sync_copy(r0.at[pl.ds(0, zrem)],
                            acc_sh.at[pl.ds(row0 + nz_full * zsrc, zrem)])
        @pl.when(sid == _NS - 1)
        def _ztail():
            pltpu.sync_copy(r0.at[pl.ds(0, rextra)],
                            acc_sh.at[pl.ds(_NS * rpt, rextra)])
        idx_load(0)
        gather_issue(0, 0)
        plsc.subcore_barrier()

        def step(i, carry):
            for b in range(2):
                t = 2 * i + b
                bn = 1 - b
                @pl.when(t >= 1)
                def _drain():
                    scatter_wait(bn, t - 1)
                @pl.when(t + 1 < hcp)
                def _next():
                    gather_issue(bn, t + 1)
                gather_wait(b, t)
                scatter_issue(b, t)
            return carry
        for ph in range(2):
            if ph:
                scatter_wait((hcp - 1) % 2, hcp - 1)
                idx_load(1)
                gather_issue(0, 0)
            lax.fori_loop(0, hcp // 2, step, 0)
        scatter_wait((hcp - 1) % 2, hcp - 1)
        plsc.subcore_barrier()

        pltpu.sync_copy(acc_sh.at[pl.ds(row0, rpt)], out_hbm.at[cid].at[pl.ds(row0, rpt)])
        @pl.when(sid == _NS - 1)
        def _():
            pltpu.sync_copy(acc_sh.at[pl.ds(_NS * rpt, rextra)],
                            out_hbm.at[cid].at[pl.ds(_NS * rpt, rextra)])

    return scatter_kernel


@functools.lru_cache(maxsize=None)
def _make_dense(n, d_in, d, relu_out):
    def body(h_ref, a0_ref, a1_ref, w1_ref, b1_ref, w2_ref, b2_ref,
             g_ref, bt_ref, o_ref):
        z = h_ref[...] + a0_ref[...] + a1_ref[...]
        z = jnp.dot(z, w1_ref[...], preferred_element_type=jnp.float32) + b1_ref[...]
        z = jnp.maximum(z, 0.0)
        z = jnp.dot(z, w2_ref[...], preferred_element_type=jnp.float32) + b2_ref[...]
        mu = jnp.mean(z, axis=0, keepdims=True)
        var = jnp.mean((z - mu) * (z - mu), axis=0, keepdims=True)
        z = g_ref[...] * (z - mu) * lax.rsqrt(var + 1e-5) + bt_ref[...]
        if relu_out:
            z = jnp.maximum(z, 0.0)
        o_ref[...] = z

    return pl.pallas_call(
        body,
        out_shape=jax.ShapeDtypeStruct((n, d), jnp.float32),
    )


def kernel(nodes, edge_indexs, graph_indicators,
           W1_0, b1_0, W2_0, b2_0, gamma_0, beta_0,
           W1_1, b1_1, W2_1, b2_1, gamma_1, beta_1,
           W1_2, b1_2, W2_2, b2_2, gamma_2, beta_2):
    del graph_indicators
    n, d = nodes.shape
    e = edge_indexs.shape[1]
    chunk, nw = 128, _NC * _NS
    q = -(-e // (nw * chunk))
    cpw = -(-q // 16) * 16
    e_pad = nw * cpw * chunk
    npad = e_pad - e
    src = jnp.concatenate([edge_indexs[0], jnp.zeros((npad,), jnp.int32)])
    dst = jnp.concatenate(
        [edge_indexs[1], jnp.full((npad,), n, jnp.int32)]).reshape(-1, chunk)
    params = [
        (W1_0, b1_0, W2_0, b2_0, gamma_0, beta_0),
        (W1_1, b1_1, W2_1, b2_1, gamma_1, beta_1),
        (W1_2, b1_2, W2_2, b2_2, gamma_2, beta_2),
    ]
    scatter = _make_scatter(n, d, e_pad)
    h = nodes
    for layer, (w1, b1, w2, b2, g, bt) in enumerate(params):
        agg = scatter(h, src, dst)
        dense = _make_dense(n, w1.shape[0], d, layer < len(params) - 1)
        h = dense(h, agg[0], agg[1], w1, b1.reshape(1, d), w2, b2.reshape(1, d),
                  g.reshape(1, d), bt.reshape(1, d))
    return h
